# Initial kernel scaffold; baseline (speedup 1.0000x reference)
import jax
import jax.numpy as jnp
from jax.experimental import pallas as pl


def kernel(x, lut):
    raise NotImplementedError("write your pallas kernel here")


# speedup vs baseline: 4.5689x; 4.5689x over previous
import functools
import math

import jax
import jax.numpy as jnp
from jax import lax
from jax.experimental import pallas as pl
from jax.experimental.pallas import tpu as pltpu, tpu_sc as plsc

N_FEATURES = 32
SCALE = math.sqrt(N_FEATURES)

NUM_CORES = 2
NUM_SUBCORES = 16
NUM_WORKERS = NUM_CORES * NUM_SUBCORES

IDX_PER_STREAM = 128
STREAMS_PER_CHUNK = 8
CHUNK = IDX_PER_STREAM * STREAMS_PER_CHUNK


def _emb_kernel(n_chunks, x_hbm, lut_hbm, out_hbm, idx_v, rows_v, sem):
    wid = lax.axis_index("s") * NUM_CORES + lax.axis_index("c")
    chunk_rows_base = wid * n_chunks

    @pl.loop(0, n_chunks)
    def _chunk(g):
        xrow = (chunk_rows_base + g) * STREAMS_PER_CHUNK
        pltpu.sync_copy(x_hbm.at[pl.ds(xrow, STREAMS_PER_CHUNK)], idx_v)
        descs = []
        for j in range(STREAMS_PER_CHUNK):
            descs.append(
                pltpu.async_copy(
                    lut_hbm.at[idx_v.at[j]],
                    rows_v.at[pl.ds(j * IDX_PER_STREAM, IDX_PER_STREAM)],
                    sem,
                )
            )
        for d in descs:
            d.wait()

        @pl.loop(0, CHUNK, unroll=8)
        def _scale(i):
            for j in range(N_FEATURES // 16):
                sl = pl.ds(j * 16, 16)
                rows_v[i, sl] = rows_v[i, sl] * SCALE

        out_row = (chunk_rows_base + g) * CHUNK
        pltpu.sync_copy(rows_v, out_hbm.at[pl.ds(out_row, CHUNK)])


@jax.jit
def _embedding_lookup(x2d, lut):
    b_total = x2d.shape[0] * x2d.shape[1]
    n_chunks = b_total // (NUM_WORKERS * CHUNK)
    mesh = plsc.VectorSubcoreMesh(
        core_axis_name="c", subcore_axis_name="s",
        num_cores=NUM_CORES, num_subcores=NUM_SUBCORES,
    )
    run = pl.kernel(
        functools.partial(_emb_kernel, n_chunks),
        out_type=jax.ShapeDtypeStruct((b_total, N_FEATURES), jnp.float32),
        mesh=mesh,
        scratch_types=[
            pltpu.VMEM((STREAMS_PER_CHUNK, IDX_PER_STREAM), jnp.int32),
            pltpu.VMEM((CHUNK, N_FEATURES), jnp.float32),
            pltpu.SemaphoreType.DMA,
        ],
        compiler_params=pltpu.CompilerParams(use_tc_tiling_on_sc=False),
    )
    return run(x2d, lut)


def kernel(x, lut):
    orig_shape = x.shape
    x2d = x.reshape(-1, IDX_PER_STREAM)
    out = _embedding_lookup(x2d, lut)
    return out.reshape(*orig_shape, N_FEATURES)


# speedup vs baseline: 4.9113x; 1.0749x over previous
import functools
import math

import jax
import jax.numpy as jnp
from jax import lax
from jax.experimental import pallas as pl
from jax.experimental.pallas import tpu as pltpu, tpu_sc as plsc

N_FEATURES = 32
SCALE = math.sqrt(N_FEATURES)

NUM_CORES = 2
NUM_SUBCORES = 16
NUM_WORKERS = NUM_CORES * NUM_SUBCORES

IDX_PER_STREAM = 128
STREAMS_PER_CHUNK = 8
CHUNK = IDX_PER_STREAM * STREAMS_PER_CHUNK


def _emb_kernel(n_chunks, x_hbm, lut_hbm, out_hbm,
                idx0, idx1, rows0, rows1, gsem0, gsem1, wsem0, wsem1):
    wid = lax.axis_index("s") * NUM_CORES + lax.axis_index("c")
    chunk_base = wid * n_chunks
    idx = (idx0, idx1)
    rows = (rows0, rows1)
    gsem = (gsem0, gsem1)
    wsem = (wsem0, wsem1)

    def fire(g, s):
        xrow = (chunk_base + g) * STREAMS_PER_CHUNK
        pltpu.sync_copy(x_hbm.at[pl.ds(xrow, STREAMS_PER_CHUNK)], idx[s])
        for j in range(STREAMS_PER_CHUNK):
            pltpu.async_copy(
                lut_hbm.at[idx[s].at[j]],
                rows[s].at[pl.ds(j * IDX_PER_STREAM, IDX_PER_STREAM)],
                gsem[s],
            )

    def drain_gathers(s):
        for j in range(STREAMS_PER_CHUNK):
            pltpu.make_async_copy(
                lut_hbm.at[idx[s].at[j]],
                rows[s].at[pl.ds(j * IDX_PER_STREAM, IDX_PER_STREAM)],
                gsem[s],
            ).wait()

    def drain_write(s):
        pltpu.make_async_copy(
            rows[s], out_hbm.at[pl.ds(0, CHUNK)], wsem[s]
        ).wait()

    def process(g, s):
        drain_gathers(s)

        @pl.loop(0, CHUNK, unroll=8)
        def _scale(i):
            for j in range(N_FEATURES // 16):
                sl = pl.ds(j * 16, 16)
                rows[s][i, sl] = rows[s][i, sl] * SCALE

        out_row = (chunk_base + g) * CHUNK
        pltpu.async_copy(rows[s], out_hbm.at[pl.ds(out_row, CHUNK)], wsem[s])

    fire(0, 0)

    @pl.loop(0, n_chunks // 2)
    def _pair(gi):
        a = 2 * gi

        @pl.when(gi > 0)
        def _():
            drain_write(1)

        fire(a + 1, 1)
        process(a, 0)

        @pl.when(a + 2 < n_chunks)
        def _():
            drain_write(0)
            fire(a + 2, 0)

        process(a + 1, 1)

    drain_write(0)
    drain_write(1)


@jax.jit
def _embedding_lookup(x2d, lut):
    b_total = x2d.shape[0] * x2d.shape[1]
    n_chunks = b_total // (NUM_WORKERS * CHUNK)
    mesh = plsc.VectorSubcoreMesh(
        core_axis_name="c", subcore_axis_name="s",
        num_cores=NUM_CORES, num_subcores=NUM_SUBCORES,
    )
    run = pl.kernel(
        functools.partial(_emb_kernel, n_chunks),
        out_type=jax.ShapeDtypeStruct((b_total, N_FEATURES), jnp.float32),
        mesh=mesh,
        scratch_types=[
            pltpu.VMEM((STREAMS_PER_CHUNK, IDX_PER_STREAM), jnp.int32),
            pltpu.VMEM((STREAMS_PER_CHUNK, IDX_PER_STREAM), jnp.int32),
            pltpu.VMEM((CHUNK, N_FEATURES), jnp.float32),
            pltpu.VMEM((CHUNK, N_FEATURES), jnp.float32),
            pltpu.SemaphoreType.DMA,
            pltpu.SemaphoreType.DMA,
            pltpu.SemaphoreType.DMA,
            pltpu.SemaphoreType.DMA,
        ],
        compiler_params=pltpu.CompilerParams(use_tc_tiling_on_sc=False),
    )
    return run(x2d, lut)


def kernel(x, lut):
    orig_shape = x.shape
    x2d = x.reshape(-1, IDX_PER_STREAM)
    out = _embedding_lookup(x2d, lut)
    return out.reshape(*orig_shape, N_FEATURES)


# speedup vs baseline: 4.9175x; 1.0013x over previous
import functools
import math

import jax
import jax.numpy as jnp
from jax import lax
from jax.experimental import pallas as pl
from jax.experimental.pallas import tpu as pltpu, tpu_sc as plsc

N_FEATURES = 32
SCALE = math.sqrt(N_FEATURES)

NUM_CORES = 2
NUM_SUBCORES = 16
NUM_WORKERS = NUM_CORES * NUM_SUBCORES

IDX_PER_STREAM = 1024
STREAMS_PER_CHUNK = 1
CHUNK = IDX_PER_STREAM * STREAMS_PER_CHUNK


def _emb_kernel(n_chunks, x_hbm, lut_hbm, out_hbm,
                idx0, idx1, rows0, rows1, gsem0, gsem1, wsem0, wsem1):
    wid = lax.axis_index("s") * NUM_CORES + lax.axis_index("c")
    chunk_base = wid * n_chunks
    idx = (idx0, idx1)
    rows = (rows0, rows1)
    gsem = (gsem0, gsem1)
    wsem = (wsem0, wsem1)

    def fire(g, s):
        xrow = (chunk_base + g) * STREAMS_PER_CHUNK
        pltpu.sync_copy(x_hbm.at[pl.ds(xrow, STREAMS_PER_CHUNK)], idx[s])
        for j in range(STREAMS_PER_CHUNK):
            pltpu.async_copy(
                lut_hbm.at[idx[s].at[j]],
                rows[s].at[pl.ds(j * IDX_PER_STREAM, IDX_PER_STREAM)],
                gsem[s],
            )

    def drain_gathers(s):
        for j in range(STREAMS_PER_CHUNK):
            pltpu.make_async_copy(
                lut_hbm.at[idx[s].at[j]],
                rows[s].at[pl.ds(j * IDX_PER_STREAM, IDX_PER_STREAM)],
                gsem[s],
            ).wait()

    def drain_write(s):
        pltpu.make_async_copy(
            rows[s], out_hbm.at[pl.ds(0, CHUNK)], wsem[s]
        ).wait()

    def process(g, s):
        drain_gathers(s)

        @pl.loop(0, CHUNK, unroll=8)
        def _scale(i):
            for j in range(N_FEATURES // 16):
                sl = pl.ds(j * 16, 16)
                rows[s][i, sl] = rows[s][i, sl] * SCALE

        out_row = (chunk_base + g) * CHUNK
        pltpu.async_copy(rows[s], out_hbm.at[pl.ds(out_row, CHUNK)], wsem[s])

    fire(0, 0)

    @pl.loop(0, n_chunks // 2)
    def _pair(gi):
        a = 2 * gi

        @pl.when(gi > 0)
        def _():
            drain_write(1)

        fire(a + 1, 1)
        process(a, 0)

        @pl.when(a + 2 < n_chunks)
        def _():
            drain_write(0)
            fire(a + 2, 0)

        process(a + 1, 1)

    drain_write(0)
    drain_write(1)


@jax.jit
def _embedding_lookup(x2d, lut):
    b_total = x2d.shape[0] * x2d.shape[1]
    n_chunks = b_total // (NUM_WORKERS * CHUNK)
    mesh = plsc.VectorSubcoreMesh(
        core_axis_name="c", subcore_axis_name="s",
        num_cores=NUM_CORES, num_subcores=NUM_SUBCORES,
    )
    run = pl.kernel(
        functools.partial(_emb_kernel, n_chunks),
        out_type=jax.ShapeDtypeStruct((b_total, N_FEATURES), jnp.float32),
        mesh=mesh,
        scratch_types=[
            pltpu.VMEM((STREAMS_PER_CHUNK, IDX_PER_STREAM), jnp.int32),
            pltpu.VMEM((STREAMS_PER_CHUNK, IDX_PER_STREAM), jnp.int32),
            pltpu.VMEM((CHUNK, N_FEATURES), jnp.float32),
            pltpu.VMEM((CHUNK, N_FEATURES), jnp.float32),
            pltpu.SemaphoreType.DMA,
            pltpu.SemaphoreType.DMA,
            pltpu.SemaphoreType.DMA,
            pltpu.SemaphoreType.DMA,
        ],
        compiler_params=pltpu.CompilerParams(use_tc_tiling_on_sc=False),
    )
    return run(x2d, lut)


def kernel(x, lut):
    orig_shape = x.shape
    x2d = x.reshape(-1, IDX_PER_STREAM)
    out = _embedding_lookup(x2d, lut)
    return out.reshape(*orig_shape, N_FEATURES)
